# Initial kernel scaffold; baseline (speedup 1.0000x reference)
import jax
import jax.numpy as jnp
from jax.experimental import pallas as pl


def kernel(x, src, dst, eattr, alpha, w_tp1, w_tp2, w_tp3, w_tp4, w_self_s, w_self_v, w_pre_s, w_pre_v, w_post_s, w_post_v, ln_gamma, ln_beta, g_w1, g_b1, g_w2, g_b2):
    raise NotImplementedError("write your pallas kernel here")


# speedup vs baseline: 3.4823x; 3.4823x over previous
import math

import jax
import jax.numpy as jnp
from jax import lax
from jax.experimental import pallas as pl
from jax.experimental.pallas import tpu as pltpu
from jax.experimental.pallas import tpu_sc as plsc

N_NODES = 100000
N_EDGES = 3200000
MUL0, MUL1 = 16, 3
D = 25

NC, NS = 2, 16
W = NC * NS
EPT = N_EDGES // W
SCK = 800
NSC = EPT // SCK
GCH = [(0, 128), (128, 128), (256, 128), (384, 128),
       (512, 128), (640, 128), (768, 32)]

SCK3 = 640
NPT = 3128
ACC = NPT * D
ACCP = 78208

_S2 = math.sqrt(2.0)
_F32 = jnp.float32
_I32 = jnp.int32

_SC_PARAMS = pltpu.CompilerParams(
    needs_layout_passes=False, use_tc_tiling_on_sc=False)


def _k1_body(x_ref, w1_ref, w2_ref, o_ref):
    xb = x_ref[...]
    xs = xb[:, :MUL0]
    a = jnp.dot(xs, w1_ref[...], preferred_element_type=_F32)
    b = jnp.dot(xs, w2_ref[...], preferred_element_type=_F32)
    xv = xb[:, MUL0:D]
    pad = jnp.zeros((xb.shape[0], 32 - 28), _F32)
    o_ref[...] = jnp.concatenate([a, b, xv, pad], axis=-1)


def _precompute_g(x, w1s, w2s):
    br = 1000
    return pl.pallas_call(
        _k1_body,
        grid=(N_NODES // br,),
        in_specs=[
            pl.BlockSpec((br, D), lambda i: (i, 0)),
            pl.BlockSpec((MUL0, MUL0), lambda i: (0, 0)),
            pl.BlockSpec((MUL0, MUL1), lambda i: (0, 0)),
        ],
        out_specs=pl.BlockSpec((br, 32), lambda i: (i, 0)),
        out_shape=jax.ShapeDtypeStruct((N_NODES, 32), _F32),
    )(x, w1s, w2s)


def _k2_body(g_hbm, src_hbm, ep_hbm, wb_hbm, mij_hbm,
             s0, s1, e0b, e1b, g0, g1, m0, m1, wv,
             ss0, ss1, se0, se1, sg0, sg1, sw0, sw1):
    wid = lax.axis_index("s") * NC + lax.axis_index("c")
    base = wid * EPT
    pltpu.sync_copy(wb_hbm, wv)
    SB = [s0, s1]
    EB = [e0b, e1b]
    GB = [g0, g1]
    MB = [m0, m1]
    SS = [ss0, ss1]
    SE = [se0, se1]
    SG = [sg0, sg1]
    SW = [sw0, sw1]
    iota = lax.broadcasted_iota(_I32, (16,), 0)

    def wrow(r):
        return wv[pl.ds(r * 16, 16)]

    def issue_src(k, b):
        e0 = base + k * SCK
        pltpu.async_copy(src_hbm.at[pl.ds(e0, SCK)], SB[b], SS[b])

    def issue_ep(k, b):
        e0 = base + k * SCK
        pltpu.async_copy(ep_hbm.at[pl.ds(e0 * 8, SCK * 8)], EB[b], SE[b])

    def gfire(b):
        pltpu.make_async_copy(src_hbm.at[pl.ds(0, SCK)], SB[b], SS[b]).wait()
        for (o, c) in GCH:
            pltpu.async_copy(
                g_hbm.at[SB[b].at[pl.ds(o, c)]],
                GB[b].at[pl.ds(o, c)], SG[b])

    def compute(b):
        gv = GB[b]
        ev_ = EB[b]
        mv = MB[b]

        def pairbody(gp, _):
            for half in range(2):
                g = gp * 2 + half
                idx16 = iota + g * 16
                idx8 = idx16 * 8
                idx25 = idx16 * 25
                ev = [plsc.load_gather(ev_, [idx8 + i]) for i in range(3)]
                al = plsc.load_gather(ev_, [idx8 + 3])
                esal = plsc.load_gather(ev_, [idx8 + 4])
                eva = [plsc.load_gather(ev_, [idx8 + 5 + i]) for i in range(3)]

                def col(c):
                    return plsc.load_gather(
                        gv, [idx16, jnp.full((16,), c, _I32)])

                xv = [col(19 + j) for j in range(9)]
                dot = [xv[u * 3] * ev[0] + xv[u * 3 + 1] * ev[1]
                       + xv[u * 3 + 2] * ev[2] for u in range(3)]
                dota = [dot[u] * al for u in range(3)]
                for f in range(MUL0):
                    t = col(f) * esal
                    for u in range(3):
                        t = t + dota[u] * wrow(u * 16 + f)
                    plsc.store_scatter(mv, [idx25 + f], t)
                for v in range(3):
                    bv = col(16 + v)
                    w3r = [wrow(48 + u * 3 + v) for u in range(3)]
                    for i in range(3):
                        p3 = (xv[i] * w3r[0] + xv[3 + i] * w3r[1]
                              + xv[6 + i] * w3r[2])
                        t = bv * eva[i] + p3 * esal
                        plsc.store_scatter(mv, [idx25 + 16 + v * 3 + i], t)
            return ()

        lax.fori_loop(0, SCK // 32, pairbody, (), unroll=False)

    def consume(k, b):
        e0 = base + k * SCK
        for (o, c) in GCH:
            pltpu.make_async_copy(
                g_hbm.at[pl.ds(0, c)], GB[b].at[pl.ds(o, c)], SG[b]).wait()

        @pl.when(k + 2 < NSC)
        def _():
            issue_src(k + 2, b)

        pltpu.make_async_copy(
            ep_hbm.at[pl.ds(0, SCK * 8)], EB[b], SE[b]).wait()

        @pl.when(k >= 2)
        def _():
            pltpu.make_async_copy(
                MB[b], mij_hbm.at[pl.ds(0, SCK * 25)], SW[b]).wait()

        compute(b)
        pltpu.async_copy(MB[b], mij_hbm.at[pl.ds(e0 * 25, SCK * 25)], SW[b])

        @pl.when(k + 2 < NSC)
        def _():
            issue_ep(k + 2, b)
            gfire(b)

    issue_src(0, 0)
    issue_ep(0, 0)
    gfire(0)
    issue_src(1, 1)
    issue_ep(1, 1)
    gfire(1)

    def pair(kk, _):
        for b in range(2):
            k = kk * 2 + b

            @pl.when(k < NSC)
            def _():
                consume(k, b)
        return ()

    lax.fori_loop(0, (NSC + 1) // 2, pair, (), unroll=False)
    for b in range(2):
        pltpu.make_async_copy(
            MB[b], mij_hbm.at[pl.ds(0, SCK * 25)], SW[b]).wait()


def _edge_messages(g, src, epack, wb):
    mesh = plsc.VectorSubcoreMesh(core_axis_name="c", subcore_axis_name="s")
    fn = pl.kernel(
        _k2_body,
        out_type=jax.ShapeDtypeStruct((N_EDGES * 25,), _F32),
        mesh=mesh,
        scratch_types=[
            pltpu.VMEM((SCK,), _I32), pltpu.VMEM((SCK,), _I32),
            pltpu.VMEM((SCK * 8,), _F32), pltpu.VMEM((SCK * 8,), _F32),
            pltpu.VMEM((SCK, 32), _F32), pltpu.VMEM((SCK, 32), _F32),
            pltpu.VMEM((SCK * 25,), _F32), pltpu.VMEM((SCK * 25,), _F32),
            pltpu.VMEM((1024,), _F32),
            pltpu.SemaphoreType.DMA, pltpu.SemaphoreType.DMA,
            pltpu.SemaphoreType.DMA, pltpu.SemaphoreType.DMA,
            pltpu.SemaphoreType.DMA, pltpu.SemaphoreType.DMA,
            pltpu.SemaphoreType.DMA, pltpu.SemaphoreType.DMA,
        ],
        compiler_params=_SC_PARAMS,
    )
    return fn(g, src, epack, wb)


def _k3_body(mij_hbm, dst_hbm, bnd_hbm, m_hbm,
             acc, m0, m1, d0, d1, b_v,
             sm0, sm1, sd0, sd1):
    wid = lax.axis_index("s") * NC + lax.axis_index("c")
    iota = lax.broadcasted_iota(_I32, (16,), 0)
    zero16 = jnp.zeros((16,), _F32)
    MB = [m0, m1]
    DB = [d0, d1]
    SM = [sm0, sm1]
    SD = [sd0, sd1]

    def zb(i, _):
        acc[pl.ds(i * 16, 16)] = zero16
        return ()

    lax.fori_loop(0, ACCP // 16, zb, (), unroll=False)

    pltpu.sync_copy(bnd_hbm, b_v)
    lo = jnp.max(plsc.load_gather(b_v, [jnp.full((16,), wid, _I32)]))
    hi = jnp.max(plsc.load_gather(b_v, [jnp.full((16,), wid + 1, _I32)]))
    c0 = lo // SCK3
    nch = (hi + SCK3 - 1) // SCK3 - c0
    nbase = wid * NPT

    def issue(k, b):
        e0 = (c0 + k) * SCK3
        pltpu.async_copy(mij_hbm.at[pl.ds(e0 * 25, SCK3 * 25)], MB[b], SM[b])
        pltpu.async_copy(dst_hbm.at[pl.ds(e0, SCK3)], DB[b], SD[b])

    def consume(k, b):
        e0 = (c0 + k) * SCK3
        pltpu.make_async_copy(
            mij_hbm.at[pl.ds(0, SCK3 * 25)], MB[b], SM[b]).wait()
        pltpu.make_async_copy(
            dst_hbm.at[pl.ds(0, SCK3)], DB[b], SD[b]).wait()

        def pairbody(gp, _):
            for half in range(2):
                g = gp * 2 + half
                idx16 = iota + g * 16
                idx25 = idx16 * 25
                eids = idx16 + e0
                valid = (eids >= lo) & (eids < hi)
                row = plsc.load_gather(DB[b], [idx16]) - nbase
                row = jnp.minimum(jnp.maximum(row, 0), NPT - 1)
                rbase = row * 25
                for f in range(D):
                    val = plsc.load_gather(MB[b], [idx25 + f])
                    plsc.addupdate_scatter(acc, [rbase + f], val, mask=valid)
            return ()

        lax.fori_loop(0, SCK3 // 32, pairbody, (), unroll=False)

        @pl.when(k + 2 < nch)
        def _():
            issue(k + 2, b)

    @pl.when(nch > 0)
    def _():
        issue(0, 0)

    @pl.when(nch > 1)
    def _():
        issue(1, 1)

    def pair(kk, _):
        for b in range(2):
            k = kk * 2 + b

            @pl.when(k < nch)
            def _():
                consume(k, b)
        return ()

    lax.fori_loop(0, (nch + 1) // 2, pair, (), unroll=False)
    pltpu.sync_copy(acc.at[pl.ds(0, ACC)], m_hbm.at[pl.ds(wid * ACC, ACC)])


def _aggregate(mij_flat, dst, bounds):
    mesh = plsc.VectorSubcoreMesh(core_axis_name="c", subcore_axis_name="s")
    fn = pl.kernel(
        _k3_body,
        out_type=jax.ShapeDtypeStruct((W * ACC,), _F32),
        mesh=mesh,
        scratch_types=[
            pltpu.VMEM((ACCP,), _F32),
            pltpu.VMEM((SCK3 * 25,), _F32), pltpu.VMEM((SCK3 * 25,), _F32),
            pltpu.VMEM((SCK3,), _I32), pltpu.VMEM((SCK3,), _I32),
            pltpu.VMEM((48,), _I32),
            pltpu.SemaphoreType.DMA, pltpu.SemaphoreType.DMA,
            pltpu.SemaphoreType.DMA, pltpu.SemaphoreType.DMA,
        ],
        compiler_params=_SC_PARAMS,
    )
    return fn(mij_flat, dst, bounds)


def _k4_body(x_ref, m_ref, wss_ref, wvs_ref, wps_ref, wvp_ref, wos_ref,
             wvo_ref, lng_ref, lnb_ref, gw1_ref, gb1_ref, gw2_ref, gb2_ref,
             r9_ref, s9_ref, o_ref):
    xb = x_ref[...]
    mb = m_ref[...]
    mm = lambda a, b: jnp.dot(a, b, preferred_element_type=_F32)
    hs = mm(xb[:, :MUL0], wss_ref[...]) + mb[:, :MUL0]
    hv = mm(xb[:, MUL0:D], wvs_ref[...]) + mb[:, MUL0:D]
    pre_s = mm(hs, wps_ref[...])
    pre_v = mm(hv, wvp_ref[...])
    s_act = jax.nn.silu(pre_s[:, :MUL0])
    gates = jax.nn.sigmoid(pre_s[:, MUL0:MUL0 + MUL1])
    vg = pre_v * mm(gates, r9_ref[...])
    post_s = mm(s_act, wos_ref[...])
    post_v = mm(vg, wvo_ref[...])
    mu = jnp.mean(post_s, axis=-1, keepdims=True)
    var = jnp.mean((post_s - mu) * (post_s - mu), axis=-1, keepdims=True)
    s_ln = (post_s - mu) * lax.rsqrt(var + 1e-5) * lng_ref[...] + lnb_ref[...]
    v_norm = jnp.sqrt(mm(post_v * post_v, s9_ref[...]) + 1e-12)
    h1 = jax.nn.silu(mm(v_norm, gw1_ref[...]) + gb1_ref[...])
    ds_ = mm(h1, gw2_ref[...]) + gb2_ref[...]
    o_ref[...] = jnp.concatenate([s_ln + ds_, post_v], axis=-1)


def _node_pipeline(x, m2d, weights):
    br = 1000
    full = lambda s: pl.BlockSpec(s, lambda i: (0, 0))
    wspecs = [full(w.shape) for w in weights]
    return pl.pallas_call(
        _k4_body,
        grid=(N_NODES // br,),
        in_specs=[
            pl.BlockSpec((br, D), lambda i: (i, 0)),
            pl.BlockSpec((br, D), lambda i: (i, 0)),
        ] + wspecs,
        out_specs=pl.BlockSpec((br, D), lambda i: (i, 0)),
        out_shape=jax.ShapeDtypeStruct((N_NODES, D), _F32),
    )(x, m2d, *weights)


def kernel(x, src, dst, eattr, alpha, w_tp1, w_tp2, w_tp3, w_tp4,
           w_self_s, w_self_v, w_pre_s, w_pre_v, w_post_s, w_post_v,
           ln_gamma, ln_beta, g_w1, g_b1, g_w2, g_b2):
    f32 = _F32
    s3 = math.sqrt(3.0)

    w1s = (w_tp1 / (4.0 * _S2)).astype(f32)
    w2s = (w_tp2 / (4.0 * _S2)).astype(f32)
    w4p = (w_tp4 / (3.0 * _S2)).astype(f32)
    w3p = (w_tp3 / (s3 * _S2)).astype(f32)
    wb = jnp.concatenate([
        jnp.broadcast_to(w4p.reshape(48, 1), (48, 16)),
        jnp.broadcast_to(w3p.reshape(9, 1), (9, 16)),
        jnp.zeros((7, 16), f32),
    ], axis=0).reshape(1024)

    eye3 = jnp.eye(3, dtype=f32)
    kron3 = lambda w: jnp.kron(w, eye3)
    k4_weights = [
        (w_self_s / 4.0).astype(f32),
        kron3(w_self_v / s3).astype(f32),
        (w_pre_s / 4.0).astype(f32),
        kron3(w_pre_v / s3).astype(f32),
        (w_post_s / 4.0).astype(f32),
        kron3(w_post_v / s3).astype(f32),
        ln_gamma.reshape(1, MUL0).astype(f32),
        ln_beta.reshape(1, MUL0).astype(f32),
        g_w1.astype(f32),
        g_b1.reshape(1, MUL1).astype(f32),
        g_w2.astype(f32),
        g_b2.reshape(1, MUL0).astype(f32),
        jnp.kron(eye3, jnp.ones((1, 3), f32)),
        jnp.kron(eye3, jnp.ones((3, 1), f32)),
    ]

    es = eattr[:, 0]
    ev0, ev1, ev2 = eattr[:, 1], eattr[:, 2], eattr[:, 3]
    al = alpha[:, 0]
    epack = jnp.stack(
        [ev0, ev1, ev2, al, es * al, ev0 * al, ev1 * al, ev2 * al],
        axis=-1).reshape(N_EDGES * 8)

    marks = jnp.arange(0, NPT * (W + 1), NPT, dtype=_I32)
    bounds = jnp.searchsorted(dst, marks, side="left").astype(_I32)
    bounds = jnp.concatenate(
        [bounds, jnp.full((15,), N_EDGES, _I32)])

    g = _precompute_g(x, w1s, w2s)
    mij_flat = _edge_messages(g, src, epack, wb)
    m_flat = _aggregate(mij_flat, dst, bounds)
    m2d = m_flat.reshape(W * NPT, D)[:N_NODES]
    x_out = _node_pipeline(x, m2d, k4_weights)
    return x_out, mij_flat.reshape(N_EDGES, D)


# speedup vs baseline: 3.9154x; 1.1244x over previous
import math

import jax
import jax.numpy as jnp
from jax import lax
from jax.experimental import pallas as pl
from jax.experimental.pallas import tpu as pltpu
from jax.experimental.pallas import tpu_sc as plsc

N_NODES = 100000
N_EDGES = 3200000
MUL0, MUL1 = 16, 3
D = 25

NC, NS = 2, 16
W = NC * NS
EPT = N_EDGES // W
SCK = 800
NSC = EPT // SCK
GCH = [(0, 128), (128, 128), (256, 128), (384, 128),
       (512, 128), (640, 128), (768, 32)]

GW = 32
EPW = 8
SCK3 = 640
NPT = 3128
ACC = NPT * D
ACCP = 78208

_S2 = math.sqrt(2.0)
_F32 = jnp.float32
_I32 = jnp.int32

_SC_PARAMS = pltpu.CompilerParams(
    needs_layout_passes=False, use_tc_tiling_on_sc=False)


def _k1_body(x_ref, w1_ref, w2_ref, o_ref):
    xb = x_ref[...]
    xs = xb[:, :MUL0]
    a = jnp.dot(xs, w1_ref[...], preferred_element_type=_F32)
    b = jnp.dot(xs, w2_ref[...], preferred_element_type=_F32)
    xv = xb[:, MUL0:D]
    pad = jnp.zeros((xb.shape[0], GW - 28), _F32)
    o_ref[...] = jnp.concatenate([a, b, xv, pad], axis=-1)


def _precompute_g(x, w1s, w2s):
    br = 1000
    return pl.pallas_call(
        _k1_body,
        grid=(N_NODES // br,),
        in_specs=[
            pl.BlockSpec((br, D), lambda i: (i, 0)),
            pl.BlockSpec((MUL0, MUL0), lambda i: (0, 0)),
            pl.BlockSpec((MUL0, MUL1), lambda i: (0, 0)),
        ],
        out_specs=pl.BlockSpec((br, GW), lambda i: (i, 0)),
        out_shape=jax.ShapeDtypeStruct((N_NODES, GW), _F32),
    )(x, w1s, w2s)


def _k2_body(g_hbm, src_hbm, ep_hbm, wb_hbm, mij_hbm,
             s0, s1, e0b, e1b, g0, g1, m0, m1, wv,
             ss0, ss1, se0, se1, sg0, sg1, sw0, sw1):
    wid = lax.axis_index("s") * NC + lax.axis_index("c")
    base = wid * EPT
    pltpu.sync_copy(wb_hbm, wv)
    SB = [s0, s1]
    EB = [e0b, e1b]
    GB = [g0, g1]
    MB = [m0, m1]
    SS = [ss0, ss1]
    SE = [se0, se1]
    SG = [sg0, sg1]
    SW = [sw0, sw1]
    iota = lax.broadcasted_iota(_I32, (16,), 0)

    def wrow(r):
        return wv[pl.ds(r * 16, 16)]

    def issue_src(k, b):
        e0 = base + k * SCK
        pltpu.async_copy(src_hbm.at[pl.ds(e0, SCK)], SB[b], SS[b])

    def issue_ep(k, b):
        e0 = base + k * SCK
        pltpu.async_copy(ep_hbm.at[pl.ds(e0 * EPW, SCK * EPW)], EB[b], SE[b])

    def gfire(b):
        pltpu.make_async_copy(src_hbm.at[pl.ds(0, SCK)], SB[b], SS[b]).wait()
        for (o, c) in GCH:
            pltpu.async_copy(
                g_hbm.at[SB[b].at[pl.ds(o, c)]],
                GB[b].at[pl.ds(o, c)], SG[b])

    def compute(b):
        gv = GB[b]
        ev_ = EB[b]
        mv = MB[b]

        def pairbody(gp, _):
            for half in range(2):
                g = gp * 2 + half
                idx16 = iota + g * 16
                idx9 = idx16 * EPW
                idx25 = idx16 * 25
                ev = [plsc.load_gather(ev_, [idx9 + i]) for i in range(3)]
                al = plsc.load_gather(ev_, [idx9 + 3])
                esal = plsc.load_gather(ev_, [idx9 + 4])
                eva = [plsc.load_gather(ev_, [idx9 + 5 + i]) for i in range(3)]

                def col(c):
                    return plsc.load_gather(
                        gv, [idx16, jnp.full((16,), c, _I32)])

                xv = [col(19 + j) for j in range(9)]
                dot = [xv[u * 3] * ev[0] + xv[u * 3 + 1] * ev[1]
                       + xv[u * 3 + 2] * ev[2] for u in range(3)]
                dota = [dot[u] * al for u in range(3)]
                for f in range(MUL0):
                    t = col(f) * esal
                    for u in range(3):
                        t = t + dota[u] * wrow(u * 16 + f)
                    plsc.store_scatter(mv, [idx25 + f], t)
                for v in range(3):
                    bv = col(16 + v)
                    w3r = [wrow(48 + u * 3 + v) for u in range(3)]
                    for i in range(3):
                        p3 = (xv[i] * w3r[0] + xv[3 + i] * w3r[1]
                              + xv[6 + i] * w3r[2])
                        t = bv * eva[i] + p3 * esal
                        plsc.store_scatter(mv, [idx25 + 16 + v * 3 + i], t)
            return ()

        lax.fori_loop(0, SCK // 32, pairbody, (), unroll=False)

    def consume(k, b):
        e0 = base + k * SCK
        for (o, c) in GCH:
            pltpu.make_async_copy(
                g_hbm.at[pl.ds(0, c)], GB[b].at[pl.ds(o, c)], SG[b]).wait()

        @pl.when(k + 2 < NSC)
        def _():
            issue_src(k + 2, b)

        pltpu.make_async_copy(
            ep_hbm.at[pl.ds(0, SCK * EPW)], EB[b], SE[b]).wait()

        @pl.when(k >= 2)
        def _():
            pltpu.make_async_copy(
                MB[b], mij_hbm.at[pl.ds(0, SCK * 25)], SW[b]).wait()

        compute(b)
        pltpu.async_copy(MB[b], mij_hbm.at[pl.ds(e0 * 25, SCK * 25)], SW[b])

        @pl.when(k + 2 < NSC)
        def _():
            issue_ep(k + 2, b)
            gfire(b)

    issue_src(0, 0)
    issue_ep(0, 0)
    gfire(0)
    issue_src(1, 1)
    issue_ep(1, 1)
    gfire(1)

    def pair(kk, _):
        for b in range(2):
            k = kk * 2 + b

            @pl.when(k < NSC)
            def _():
                consume(k, b)
        return ()

    lax.fori_loop(0, (NSC + 1) // 2, pair, (), unroll=False)
    for b in range(2):
        pltpu.make_async_copy(
            MB[b], mij_hbm.at[pl.ds(0, SCK * 25)], SW[b]).wait()


def _edge_messages(g, src, epack, wb):
    mesh = plsc.VectorSubcoreMesh(core_axis_name="c", subcore_axis_name="s")
    fn = pl.kernel(
        _k2_body,
        out_type=jax.ShapeDtypeStruct((N_EDGES * 25,), _F32),
        mesh=mesh,
        scratch_types=[
            pltpu.VMEM((SCK,), _I32), pltpu.VMEM((SCK,), _I32),
            pltpu.VMEM((SCK * EPW,), _F32), pltpu.VMEM((SCK * EPW,), _F32),
            pltpu.VMEM((SCK, GW), _F32), pltpu.VMEM((SCK, GW), _F32),
            pltpu.VMEM((SCK * 25,), _F32), pltpu.VMEM((SCK * 25,), _F32),
            pltpu.VMEM((1024,), _F32),
            pltpu.SemaphoreType.DMA, pltpu.SemaphoreType.DMA,
            pltpu.SemaphoreType.DMA, pltpu.SemaphoreType.DMA,
            pltpu.SemaphoreType.DMA, pltpu.SemaphoreType.DMA,
            pltpu.SemaphoreType.DMA, pltpu.SemaphoreType.DMA,
        ],
        compiler_params=_SC_PARAMS,
    )
    return fn(g, src, epack, wb)


def _k3_body(mij_hbm, dst_hbm, m_hbm,
             acc, m0, m1, d0, d1, b_v,
             sm0, sm1, sd0, sd1):
    wid = lax.axis_index("s") * NC + lax.axis_index("c")
    iota = lax.broadcasted_iota(_I32, (16,), 0)
    zero16 = jnp.zeros((16,), _F32)
    MB = [m0, m1]
    DB = [d0, d1]
    SM = [sm0, sm1]
    SD = [sd0, sd1]

    def zb(i, _):
        acc[pl.ds(i * 16, 16)] = zero16
        return ()

    lax.fori_loop(0, ACCP // 16, zb, (), unroll=False)

    def bsearch(target):
        def step(_, lh):
            lo_, hi_ = lh
            mid = jnp.minimum((lo_ + hi_) // 2, N_EDGES - 1)
            m16 = pl.multiple_of(mid - mid % 16, 16)
            pltpu.sync_copy(dst_hbm.at[pl.ds(m16, 16)], b_v)
            v = jnp.max(plsc.load_gather(
                b_v, [jnp.full((16,), 1, _I32) * (mid - m16)]))
            right = v < target
            return (jnp.where(right, mid + 1, lo_),
                    jnp.where(right, hi_, mid))
        lo_, _ = lax.fori_loop(0, 22, step, (0, N_EDGES))
        return lo_

    lo = bsearch(wid * NPT)
    hi = bsearch((wid + 1) * NPT)
    c0 = lo // SCK3
    nch = (hi + SCK3 - 1) // SCK3 - c0
    nbase = wid * NPT

    def issue(k, b):
        e0 = (c0 + k) * SCK3
        pltpu.async_copy(mij_hbm.at[pl.ds(e0 * 25, SCK3 * 25)], MB[b], SM[b])
        pltpu.async_copy(dst_hbm.at[pl.ds(e0, SCK3)], DB[b], SD[b])

    def consume(k, b):
        e0 = (c0 + k) * SCK3
        pltpu.make_async_copy(
            mij_hbm.at[pl.ds(0, SCK3 * 25)], MB[b], SM[b]).wait()
        pltpu.make_async_copy(
            dst_hbm.at[pl.ds(0, SCK3)], DB[b], SD[b]).wait()

        stride = SCK3 // 16
        lanes = iota * stride

        def stepbody(t, _):
            idx16 = lanes + t
            idx25 = idx16 * 25
            eids = idx16 + e0
            valid = (eids >= lo) & (eids < hi)
            row = plsc.load_gather(DB[b], [idx16]) - nbase
            row = jnp.minimum(jnp.maximum(row, 0), NPT - 1)
            rbase = row * 25
            for f in range(D):
                val = plsc.load_gather(MB[b], [idx25 + f])
                plsc.addupdate_scatter(acc, [rbase + f], val, mask=valid)
            return ()

        lax.fori_loop(0, stride, stepbody, (), unroll=False)

        @pl.when(k + 2 < nch)
        def _():
            issue(k + 2, b)

    @pl.when(nch > 0)
    def _():
        issue(0, 0)

    @pl.when(nch > 1)
    def _():
        issue(1, 1)

    def pair(kk, _):
        for b in range(2):
            k = kk * 2 + b

            @pl.when(k < nch)
            def _():
                consume(k, b)
        return ()

    lax.fori_loop(0, (nch + 1) // 2, pair, (), unroll=False)
    pltpu.sync_copy(acc.at[pl.ds(0, ACC)], m_hbm.at[pl.ds(wid * ACC, ACC)])


def _aggregate(mij_flat, dst):
    mesh = plsc.VectorSubcoreMesh(core_axis_name="c", subcore_axis_name="s")
    fn = pl.kernel(
        _k3_body,
        out_type=jax.ShapeDtypeStruct((W * ACC,), _F32),
        mesh=mesh,
        scratch_types=[
            pltpu.VMEM((ACCP,), _F32),
            pltpu.VMEM((SCK3 * 25,), _F32), pltpu.VMEM((SCK3 * 25,), _F32),
            pltpu.VMEM((SCK3,), _I32), pltpu.VMEM((SCK3,), _I32),
            pltpu.VMEM((16,), _I32),
            pltpu.SemaphoreType.DMA, pltpu.SemaphoreType.DMA,
            pltpu.SemaphoreType.DMA, pltpu.SemaphoreType.DMA,
        ],
        compiler_params=_SC_PARAMS,
    )
    return fn(mij_flat, dst)


def _k4_body(x_ref, m_ref, wss_ref, wvs_ref, wps_ref, wvp_ref, wos_ref,
             wvo_ref, lng_ref, lnb_ref, gw1_ref, gb1_ref, gw2_ref, gb2_ref,
             r9_ref, s9_ref, o_ref):
    xb = x_ref[...]
    mb = m_ref[...]
    mm = lambda a, b: jnp.dot(a, b, preferred_element_type=_F32)
    hs = mm(xb[:, :MUL0], wss_ref[...]) + mb[:, :MUL0]
    hv = mm(xb[:, MUL0:D], wvs_ref[...]) + mb[:, MUL0:D]
    pre_s = mm(hs, wps_ref[...])
    pre_v = mm(hv, wvp_ref[...])
    s_act = jax.nn.silu(pre_s[:, :MUL0])
    gates = jax.nn.sigmoid(pre_s[:, MUL0:MUL0 + MUL1])
    vg = pre_v * mm(gates, r9_ref[...])
    post_s = mm(s_act, wos_ref[...])
    post_v = mm(vg, wvo_ref[...])
    mu = jnp.mean(post_s, axis=-1, keepdims=True)
    var = jnp.mean((post_s - mu) * (post_s - mu), axis=-1, keepdims=True)
    s_ln = (post_s - mu) * lax.rsqrt(var + 1e-5) * lng_ref[...] + lnb_ref[...]
    v_norm = jnp.sqrt(mm(post_v * post_v, s9_ref[...]) + 1e-12)
    h1 = jax.nn.silu(mm(v_norm, gw1_ref[...]) + gb1_ref[...])
    ds_ = mm(h1, gw2_ref[...]) + gb2_ref[...]
    o_ref[...] = jnp.concatenate([s_ln + ds_, post_v], axis=-1)


def _node_pipeline(x, m2d, weights):
    br = 1000
    full = lambda s: pl.BlockSpec(s, lambda i: (0, 0))
    wspecs = [full(w.shape) for w in weights]
    return pl.pallas_call(
        _k4_body,
        grid=(N_NODES // br,),
        in_specs=[
            pl.BlockSpec((br, D), lambda i: (i, 0)),
            pl.BlockSpec((br, D), lambda i: (i, 0)),
        ] + wspecs,
        out_specs=pl.BlockSpec((br, D), lambda i: (i, 0)),
        out_shape=jax.ShapeDtypeStruct((N_NODES, D), _F32),
    )(x, m2d, *weights)


def kernel(x, src, dst, eattr, alpha, w_tp1, w_tp2, w_tp3, w_tp4,
           w_self_s, w_self_v, w_pre_s, w_pre_v, w_post_s, w_post_v,
           ln_gamma, ln_beta, g_w1, g_b1, g_w2, g_b2):
    f32 = _F32
    s3 = math.sqrt(3.0)

    w1s = (w_tp1 / (4.0 * _S2)).astype(f32)
    w2s = (w_tp2 / (4.0 * _S2)).astype(f32)
    w4p = (w_tp4 / (3.0 * _S2)).astype(f32)
    w3p = (w_tp3 / (s3 * _S2)).astype(f32)
    wb = jnp.concatenate([
        jnp.broadcast_to(w4p.reshape(48, 1), (48, 16)),
        jnp.broadcast_to(w3p.reshape(9, 1), (9, 16)),
        jnp.zeros((7, 16), f32),
    ], axis=0).reshape(1024)

    eye3 = jnp.eye(3, dtype=f32)
    kron3 = lambda w: jnp.kron(w, eye3)
    k4_weights = [
        (w_self_s / 4.0).astype(f32),
        kron3(w_self_v / s3).astype(f32),
        (w_pre_s / 4.0).astype(f32),
        kron3(w_pre_v / s3).astype(f32),
        (w_post_s / 4.0).astype(f32),
        kron3(w_post_v / s3).astype(f32),
        ln_gamma.reshape(1, MUL0).astype(f32),
        ln_beta.reshape(1, MUL0).astype(f32),
        g_w1.astype(f32),
        g_b1.reshape(1, MUL1).astype(f32),
        g_w2.astype(f32),
        g_b2.reshape(1, MUL0).astype(f32),
        jnp.kron(eye3, jnp.ones((1, 3), f32)),
        jnp.kron(eye3, jnp.ones((3, 1), f32)),
    ]

    es = eattr[:, 0]
    ev0, ev1, ev2 = eattr[:, 1], eattr[:, 2], eattr[:, 3]
    al = alpha[:, 0]
    epack = jnp.stack(
        [ev0, ev1, ev2, al, es * al, ev0 * al, ev1 * al, ev2 * al],
        axis=-1).reshape(N_EDGES * EPW)

    g = _precompute_g(x, w1s, w2s)
    mij_flat = _edge_messages(g, src, epack, wb)
    m_flat = _aggregate(mij_flat, dst)
    m2d = m_flat.reshape(W * NPT, D)[:N_NODES]
    x_out = _node_pipeline(x, m2d, k4_weights)
    return x_out, mij_flat.reshape(N_EDGES, D)
